# baseline (speedup 1.0000x reference)
import jax
import jax.numpy as jnp
from jax.experimental import pallas as pl
from jax.experimental.pallas import tpu as pltpu

_HB = 6400


def _mlp_block_kernel(seq0_ref, seq1_ref, w1_ref, b1_ref, w2t_ref, b2_ref,
                      w3_ref, b3_ref, out_ref):
    w1 = w1_ref[...]
    w2 = w2t_ref[...].T
    w3 = w3_ref[...]
    for k, seq_ref in enumerate((seq0_ref, seq1_ref)):
        x = seq_ref[...]
        h = jnp.dot(x, w1, preferred_element_type=jnp.float32)
        h = jnp.maximum(h + b1_ref[...], 0.0)
        h = jnp.dot(h, w2, preferred_element_type=jnp.float32)
        h = jnp.maximum(h + b2_ref[...], 0.0)
        h = jnp.dot(h, w3, preferred_element_type=jnp.float32)
        out_ref[:, k * _HB:(k + 1) * _HB] = (h + b3_ref[...]).T


def _fused_mlp(seq, W1, b1, W2t, b2, W3, b3, *, interpret=False):
    n, ft_in = seq.shape
    h1 = W1.shape[1]
    h2 = W2t.shape[0]
    nc = W3.shape[1]
    grid = (pl.cdiv(n, 2 * _HB),)
    full = lambda shape: pl.BlockSpec(shape, lambda i: (0, 0))
    return pl.pallas_call(
        _mlp_block_kernel,
        grid=grid,
        in_specs=[
            pl.BlockSpec((_HB, ft_in), lambda i: (2 * i, 0)),
            pl.BlockSpec((_HB, ft_in), lambda i: (2 * i + 1, 0)),
            full((ft_in, h1)),
            full((1, h1)),
            full((h2, h1)),
            full((1, h2)),
            full((h2, nc)),
            full((1, nc)),
        ],
        out_specs=pl.BlockSpec((nc, 2 * _HB), lambda i: (0, i)),
        out_shape=jax.ShapeDtypeStruct((nc, n), seq.dtype),
        compiler_params=pltpu.CompilerParams(
            dimension_semantics=("parallel",),
            vmem_limit_bytes=100 * 1024 * 1024,
        ),
        interpret=interpret,
    )(seq, seq, W1, b1.reshape(1, h1), W2t, b2.reshape(1, h2), W3,
      b3.reshape(1, nc))


def kernel(seq, W1, b1, W2, b2, W3, b3):
    out_t = _fused_mlp(seq, W1, b1, W2.T, b2, W3, b3)
    return out_t.T


# speedup vs baseline: 1.0728x; 1.0728x over previous
import jax
import jax.numpy as jnp
from jax.experimental import pallas as pl
from jax.experimental.pallas import tpu as pltpu

_BR = 12800


def _mlp_block_kernel(seq_ref, w1_ref, b1_ref, w2t_ref, b2_ref, w3_ref,
                      b3_ref, out_ref):
    x = seq_ref[...].astype(jnp.bfloat16)
    h = jnp.dot(x, w1_ref[...].astype(jnp.bfloat16),
                preferred_element_type=jnp.float32)
    h = jnp.maximum(h + b1_ref[...], 0.0).astype(jnp.bfloat16)
    h = jnp.dot(h, w2t_ref[...].astype(jnp.bfloat16).T,
                preferred_element_type=jnp.float32)
    h = jnp.maximum(h + b2_ref[...], 0.0).astype(jnp.bfloat16)
    h = jnp.dot(h, w3_ref[...].astype(jnp.bfloat16),
                preferred_element_type=jnp.float32)
    out_ref[...] = (h + b3_ref[...]).T


def _fused_mlp(seq, W1, b1, W2t, b2, W3, b3, *, block_rows=_BR,
               interpret=False):
    n, ft_in = seq.shape
    h1 = W1.shape[1]
    h2 = W2t.shape[0]
    nc = W3.shape[1]
    grid = (pl.cdiv(n, block_rows),)
    full = lambda shape: pl.BlockSpec(shape, lambda i: (0, 0))
    return pl.pallas_call(
        _mlp_block_kernel,
        grid=grid,
        in_specs=[
            pl.BlockSpec((block_rows, ft_in), lambda i: (i, 0)),
            full((ft_in, h1)),
            full((1, h1)),
            full((h2, h1)),
            full((1, h2)),
            full((h2, nc)),
            full((1, nc)),
        ],
        out_specs=pl.BlockSpec((nc, block_rows), lambda i: (0, i)),
        out_shape=jax.ShapeDtypeStruct((nc, n), seq.dtype),
        compiler_params=pltpu.CompilerParams(
            dimension_semantics=("parallel",),
            vmem_limit_bytes=100 * 1024 * 1024,
        ),
        interpret=interpret,
    )(seq, W1, b1.reshape(1, h1), W2t, b2.reshape(1, h2), W3,
      b3.reshape(1, nc))


def kernel(seq, W1, b1, W2, b2, W3, b3):
    out_t = _fused_mlp(seq, W1, b1, W2.T, b2, W3, b3)
    return out_t.T
